# Initial kernel scaffold; baseline (speedup 1.0000x reference)
import jax
import jax.numpy as jnp
from jax.experimental import pallas as pl


def kernel(x, edge_index, batch, params):
    raise NotImplementedError("write your pallas kernel here")


# speedup vs baseline: 14.2464x; 14.2464x over previous
import functools

import jax
import jax.numpy as jnp
from jax import lax
from jax.experimental import pallas as pl
from jax.experimental.pallas import tpu as pltpu
from jax.experimental.pallas import tpu_sc as plsc

N = 10000
E = 320000
D = 128
H = 128
C = 10
G = 128
EPS = 1e-5

NC = 2
NS = 16
NW = NC * NS
EPT = E // NW
K = 80
NCHUNK = EPT // K
NPAD = 10240
RPT = NPAD // NS
ZR = 32
DPT = NPAD // NS

@functools.cache
def _sc_mesh():
    return plsc.VectorSubcoreMesh(core_axis_name="c", subcore_axis_name="s",
                                  num_cores=NC, num_subcores=NS)


def _sc_deg_body(col_hbm, out_hbm, deg_sh, colbuf, ones_v, zbuf):
    c = lax.axis_index("c")
    s = lax.axis_index("s")
    wid = c * NS + s
    for i in range(DPT // 16):
        zbuf[pl.ds(i * 16, 16)] = jnp.zeros((16,), jnp.float32)
    for i in range(K // 16):
        ones_v[pl.ds(i * 16, 16)] = jnp.ones((16,), jnp.float32)
    pltpu.sync_copy(zbuf, deg_sh.at[pl.ds(s * DPT, DPT)])
    plsc.subcore_barrier()
    base = wid * EPT

    @pl.loop(0, NCHUNK)
    def _(j):
        pltpu.sync_copy(col_hbm.at[pl.ds(base + j * K, K)], colbuf)
        pltpu.sync_copy(ones_v, deg_sh.at[colbuf], add=True)

    plsc.subcore_barrier()
    pltpu.sync_copy(deg_sh.at[pl.ds(s * DPT, DPT)],
                    out_hbm.at[pl.ds((c * NS + s) * DPT, DPT)])


@functools.cache
def _sc_deg_kernel():
    return pl.kernel(
        _sc_deg_body,
        out_type=jax.ShapeDtypeStruct((NC * NPAD,), jnp.float32),
        mesh=_sc_mesh(),
        scratch_types=[
            pltpu.VMEM_SHARED((NPAD,), jnp.float32),
            pltpu.VMEM((K,), jnp.int32),
            pltpu.VMEM((K,), jnp.float32),
            pltpu.VMEM((DPT,), jnp.float32),
        ],
    )


def _sc_deg(col):
    return _sc_deg_kernel()(col)


def _sc_prop_body(hs_hbm, row_hbm, col_hbm, out_hbm,
                  acc_sh, rowbuf, colbuf, rows, zbuf, sem):
    c = lax.axis_index("c")
    s = lax.axis_index("s")
    wid = c * NS + s
    for r in range(ZR):
        for q in range(H // 16):
            zbuf[r, pl.ds(q * 16, 16)] = jnp.zeros((16,), jnp.float32)

    @pl.loop(0, RPT // ZR)
    def _(i):
        pltpu.sync_copy(zbuf, acc_sh.at[pl.ds(s * RPT + i * ZR, ZR)])

    plsc.subcore_barrier()
    base = wid * EPT

    @pl.loop(0, NCHUNK)
    def _(j):
        pltpu.sync_copy(row_hbm.at[pl.ds(base + j * K, K)], rowbuf)
        pltpu.sync_copy(col_hbm.at[pl.ds(base + j * K, K)], colbuf)
        pltpu.async_copy(hs_hbm.at[rowbuf], rows, sem).wait()
        pltpu.sync_copy(rows, acc_sh.at[colbuf], add=True)

    plsc.subcore_barrier()
    pltpu.sync_copy(acc_sh.at[pl.ds(s * RPT, RPT)],
                    out_hbm.at[c, pl.ds(s * RPT, RPT)])


@functools.cache
def _sc_prop_kernel():
    return pl.kernel(
        _sc_prop_body,
        out_type=jax.ShapeDtypeStruct((NC, NPAD, H), jnp.float32),
        mesh=_sc_mesh(),
        scratch_types=[
            pltpu.VMEM_SHARED((NPAD, H), jnp.float32),
            pltpu.VMEM((K,), jnp.int32),
            pltpu.VMEM((K,), jnp.int32),
            pltpu.VMEM((K, H), jnp.float32),
            pltpu.VMEM((ZR, H), jnp.float32),
            pltpu.SemaphoreType.DMA,
        ],
    )


def _sc_prop(hs, row, col):
    return _sc_prop_kernel()(hs, row, col)


def _bn(x):
    m = jnp.mean(x, axis=0, keepdims=True)
    v = jnp.mean((x - m) * (x - m), axis=0, keepdims=True)
    return (x - m) * lax.rsqrt(v + EPS) + 1e-4


def _mm(a, b):
    return jnp.dot(a, b, preferred_element_type=jnp.float32)


def _dinv_body(degp_ref, out_ref):
    d = degp_ref[0:1, :] + degp_ref[1:2, :] + 1.0
    out_ref[...] = lax.rsqrt(d)


def _tc_dinv(degp):
    return pl.pallas_call(
        _dinv_body,
        out_shape=jax.ShapeDtypeStruct((1, NPAD), jnp.float32),
    )(degp)


def _pre_body(x_ref, dinv_ref, Wf_ref, bf_ref, W0_ref, out_ref):
    xb = _bn(x_ref[...])
    x1 = jnp.maximum(_mm(xb, Wf_ref[...]) + bf_ref[...], 0.0)
    z = _bn(x1)
    out_ref[...] = _mm(z, W0_ref[...]) * dinv_ref[...]


def _tc_pre(x, dinv, Wf, bf, W0):
    return pl.pallas_call(
        _pre_body,
        out_shape=jax.ShapeDtypeStruct((N, H), jnp.float32),
    )(x, dinv, Wf, bf, W0)


def _mid_body(accp_ref, hs_ref, dinv_ref, b_ref, W_ref, out_ref):
    acc = accp_ref[0, :N, :] + accp_ref[1, :N, :] + hs_ref[...]
    x = jnp.maximum(acc * dinv_ref[...] + b_ref[...], 0.0)
    z = _bn(x)
    out_ref[...] = _mm(z, W_ref[...]) * dinv_ref[...]


def _tc_mid(accp, hs, dinv, b, W):
    return pl.pallas_call(
        _mid_body,
        out_shape=jax.ShapeDtypeStruct((N, H), jnp.float32),
    )(accp, hs, dinv, b, W)


def _att1_body(accp_ref, hs_ref, dinv_ref, b_ref, Wn_ref, bn2_ref,
               natt_ref, xc_att_ref, xo_att_ref):
    acc = accp_ref[0, :N, :] + accp_ref[1, :N, :] + hs_ref[...]
    x = jnp.maximum(acc * dinv_ref[...] + b_ref[...], 0.0)
    logits = _mm(x, Wn_ref[...]) + bn2_ref[...]
    mx = jnp.max(logits, axis=-1, keepdims=True)
    e = jnp.exp(logits - mx)
    a = e / jnp.sum(e, axis=-1, keepdims=True)
    natt_ref[...] = a
    xc_att_ref[...] = a[:, 0:1] * x
    xo_att_ref[...] = a[:, 1:2] * x


def _tc_att1(accp, hs, dinv, b, Wn, bn2):
    return pl.pallas_call(
        _att1_body,
        out_shape=(
            jax.ShapeDtypeStruct((N, 2), jnp.float32),
            jax.ShapeDtypeStruct((N, H), jnp.float32),
            jax.ShapeDtypeStruct((N, H), jnp.float32),
        ),
    )(accp, hs, dinv, b, Wn, bn2)


def _att2_body(xc_ref, xo_ref, dinv_ref, Wc_ref, Wo_ref, hcs_ref, hos_ref):
    hcs_ref[...] = _mm(_bn(xc_ref[...]), Wc_ref[...]) * dinv_ref[...]
    hos_ref[...] = _mm(_bn(xo_ref[...]), Wo_ref[...]) * dinv_ref[...]


def _tc_att2(xc_att, xo_att, dinv, Wc, Wo):
    return pl.pallas_call(
        _att2_body,
        out_shape=(
            jax.ShapeDtypeStruct((N, H), jnp.float32),
            jax.ShapeDtypeStruct((N, H), jnp.float32),
        ),
    )(xc_att, xo_att, dinv, Wc, Wo)


def _pool_body(accc_ref, acco_ref, hcs_ref, hos_ref, dinv_ref,
               bc_ref, bo_ref, batch_ref, sc_ref, so_ref):
    xc = jnp.maximum((accc_ref[0, :N, :] + accc_ref[1, :N, :] + hcs_ref[...])
                     * dinv_ref[...] + bc_ref[...], 0.0)
    xo = jnp.maximum((acco_ref[0, :N, :] + acco_ref[1, :N, :] + hos_ref[...])
                     * dinv_ref[...] + bo_ref[...], 0.0)
    gid = lax.broadcasted_iota(jnp.int32, (G, N), 0)
    onehot = jnp.where(gid == batch_ref[...], 1.0, 0.0)
    sc_ref[...] = _mm(onehot, xc)
    so_ref[...] = _mm(onehot, xo)


def _tc_pool(accc, acco, hcs, hos, dinv, bc, bo, batch_row):
    return pl.pallas_call(
        _pool_body,
        out_shape=(
            jax.ShapeDtypeStruct((G, H), jnp.float32),
            jax.ShapeDtypeStruct((G, H), jnp.float32),
        ),
    )(accc, acco, hcs, hos, dinv, bc, bo, batch_row)


def _readout(h, W1, b1, W2, b2):
    h = _bn(h)
    h = jnp.maximum(_mm(h, W1) + b1, 0.0)
    h = _bn(h)
    l = _mm(h, W2) + b2
    mx = jnp.max(l, axis=-1, keepdims=True)
    return l - mx - jnp.log(jnp.sum(jnp.exp(l - mx), axis=-1, keepdims=True))


def _head_body(sc_ref, so_ref,
               W1c_ref, b1c_ref, W2c_ref, b2c_ref,
               W1o_ref, b1o_ref, W2o_ref, b2o_ref,
               W1co_ref, b1co_ref, W2co_ref, b2co_ref,
               lc_ref, lo_ref, lco_ref):
    sc = sc_ref[...]
    so = so_ref[...]
    lc_ref[...] = _readout(sc, W1c_ref[...], b1c_ref[...],
                           W2c_ref[...], b2c_ref[...])
    lo_ref[...] = _readout(so, W1o_ref[...], b1o_ref[...],
                           W2o_ref[...], b2o_ref[...])
    sco = jnp.concatenate([sc, so], axis=1)
    lco_ref[...] = _readout(sco, W1co_ref[...], b1co_ref[...],
                            W2co_ref[...], b2co_ref[...])


def _tc_head(sc, so, *weights):
    return pl.pallas_call(
        _head_body,
        out_shape=(
            jax.ShapeDtypeStruct((G, C), jnp.float32),
            jax.ShapeDtypeStruct((G, C), jnp.float32),
            jax.ShapeDtypeStruct((G, C), jnp.float32),
        ),
    )(sc, so, *weights)


def kernel(x, edge_index, batch, params):
    p = params
    ei = edge_index

    row = ei[0]
    col = ei[1]
    degp = _sc_deg(col).reshape(NC, NPAD)
    dinv_row = _tc_dinv(degp)
    dinv = dinv_row[0, :N].reshape(N, 1)

    hs0 = _tc_pre(x, dinv, p['W_feat'], p['b_feat'].reshape(1, H), p['Ws'][0])
    acc0 = _sc_prop(hs0, row, col)
    hs1 = _tc_mid(acc0, hs0, dinv, p['bs'][0].reshape(1, H), p['Ws'][1])
    acc1 = _sc_prop(hs1, row, col)
    hs2 = _tc_mid(acc1, hs1, dinv, p['bs'][1].reshape(1, H), p['Ws'][2])
    acc2 = _sc_prop(hs2, row, col)

    Wn = jnp.concatenate([p['Wn_c'], p['Wn_o']], axis=1)
    bn2 = jnp.concatenate([p['bn_c'], p['bn_o']]).reshape(1, 2)
    node_att, xc_att, xo_att = _tc_att1(
        acc2, hs2, dinv, p['bs'][2].reshape(1, H), Wn, bn2)
    hcs, hos = _tc_att2(xc_att, xo_att, dinv, p['W_ctx'], p['W_obj'])

    accc = _sc_prop(hcs, row, col)
    acco = _sc_prop(hos, row, col)

    sc, so = _tc_pool(accc, acco, hcs, hos, dinv,
                      p['b_ctx'].reshape(1, H), p['b_obj'].reshape(1, H),
                      batch.reshape(1, N))

    xc_logis, xo_logis, xco_logis = _tc_head(
        sc, so,
        p['W1c'], p['b1c'].reshape(1, H), p['W2c'], p['b2c'].reshape(1, C),
        p['W1o'], p['b1o'].reshape(1, H), p['W2o'], p['b2o'].reshape(1, C),
        p['W1co'], p['b1co'].reshape(1, H), p['W2co'], p['b2co'].reshape(1, C))

    edge_att_c = jnp.ones((E, 1), jnp.float32)
    return (xc_logis, xo_logis, xco_logis, xc_att, xo_att,
            edge_att_c, node_att)


# speedup vs baseline: 32.9561x; 2.3133x over previous
import functools

import jax
import jax.numpy as jnp
from jax import lax
from jax.experimental import pallas as pl
from jax.experimental.pallas import tpu as pltpu
from jax.experimental.pallas import tpu_sc as plsc

N = 10000
E = 320000
D = 128
H = 128
C = 10
G = 128
EPS = 1e-5

NC = 2
NS = 16
NW = NC * NS
EPT = E // NW
K = 80
NCHUNK = EPT // K
NPAD = 10240
RPT = NPAD // NS
ZR = 32
DPT = NPAD // NS

@functools.cache
def _sc_mesh():
    return plsc.VectorSubcoreMesh(core_axis_name="c", subcore_axis_name="s",
                                  num_cores=NC, num_subcores=NS)


def _sc_deg_body(col_hbm, out_hbm, deg_sh, colbuf, ones_v, zbuf):
    c = lax.axis_index("c")
    s = lax.axis_index("s")
    wid = c * NS + s
    for i in range(DPT // 16):
        zbuf[pl.ds(i * 16, 16)] = jnp.zeros((16,), jnp.float32)
    for i in range(K // 16):
        ones_v[pl.ds(i * 16, 16)] = jnp.ones((16,), jnp.float32)
    pltpu.sync_copy(col_hbm.at[wid], colbuf)
    pltpu.sync_copy(zbuf, deg_sh.at[pl.ds(s * DPT, DPT)])
    plsc.subcore_barrier()

    @pl.loop(0, NCHUNK)
    def _(j):
        pltpu.sync_copy(ones_v, deg_sh.at[colbuf.at[j]], add=True)

    plsc.subcore_barrier()
    pltpu.sync_copy(deg_sh.at[pl.ds(s * DPT, DPT)],
                    out_hbm.at[pl.ds((c * NS + s) * DPT, DPT)])


@functools.cache
def _sc_deg_kernel():
    return pl.kernel(
        _sc_deg_body,
        out_type=jax.ShapeDtypeStruct((NC * NPAD,), jnp.float32),
        mesh=_sc_mesh(),
        scratch_types=[
            pltpu.VMEM_SHARED((NPAD,), jnp.float32),
            pltpu.VMEM((NCHUNK, K), jnp.int32),
            pltpu.VMEM((K,), jnp.float32),
            pltpu.VMEM((DPT,), jnp.float32),
        ],
    )


def _sc_deg(col):
    return _sc_deg_kernel()(col.reshape(NW, NCHUNK, K))


def _sc_prop_body(hs_hbm, row_hbm, col_hbm, out_hbm,
                  acc_sh, rowbuf, colbuf, rows0, rows1,
                  sem0, sem1):
    c = lax.axis_index("c")
    s = lax.axis_index("s")
    wid = c * NS + s
    for r in range(K):
        for q in range(H // 16):
            rows0[r, pl.ds(q * 16, 16)] = jnp.zeros((16,), jnp.float32)

    pltpu.sync_copy(row_hbm.at[wid], rowbuf)
    pltpu.sync_copy(col_hbm.at[wid], colbuf)

    @pl.loop(0, RPT // K)
    def _(i):
        pltpu.sync_copy(rows0, acc_sh.at[pl.ds(s * RPT + i * K, K)])

    plsc.subcore_barrier()

    pltpu.async_copy(hs_hbm.at[rowbuf.at[pl.ds(0, K)]], rows0, sem0).wait()

    @pl.loop(0, NCHUNK - 1, step=2)
    def _(j):
        cp1 = pltpu.async_copy(
            hs_hbm.at[rowbuf.at[pl.ds((j + 1) * K, K)]], rows1, sem1)
        pltpu.sync_copy(rows0, acc_sh.at[colbuf.at[j]], add=True)
        cp0 = pltpu.async_copy(
            hs_hbm.at[rowbuf.at[pl.ds((j + 2) * K, K)]], rows0, sem0)
        cp1.wait()
        pltpu.sync_copy(rows1, acc_sh.at[colbuf.at[j + 1]], add=True)
        cp0.wait()

    pltpu.sync_copy(rows0, acc_sh.at[colbuf.at[NCHUNK - 1]], add=True)

    plsc.subcore_barrier()
    pltpu.sync_copy(acc_sh.at[pl.ds(s * RPT, RPT)],
                    out_hbm.at[c, pl.ds(s * RPT, RPT)])


@functools.cache
def _sc_prop_kernel():
    return pl.kernel(
        _sc_prop_body,
        out_type=jax.ShapeDtypeStruct((NC, NPAD, H), jnp.float32),
        mesh=_sc_mesh(),
        scratch_types=[
            pltpu.VMEM_SHARED((NPAD, H), jnp.float32),
            pltpu.VMEM((EPT,), jnp.int32),
            pltpu.VMEM((NCHUNK, K), jnp.int32),
            pltpu.VMEM((K, H), jnp.float32),
            pltpu.VMEM((K, H), jnp.float32),
            pltpu.SemaphoreType.DMA,
            pltpu.SemaphoreType.DMA,
        ],
    )


def _sc_prop(hs, row, col):
    return _sc_prop_kernel()(hs.reshape(N, H),
                             row.reshape(NW, EPT),
                             col.reshape(NW, NCHUNK, K))


def _bn(x):
    m = jnp.mean(x, axis=0, keepdims=True)
    v = jnp.mean((x - m) * (x - m), axis=0, keepdims=True)
    return (x - m) * lax.rsqrt(v + EPS) + 1e-4


def _mm(a, b):
    return jnp.dot(a, b, preferred_element_type=jnp.float32)


def _dinv_body(degp_ref, out_ref):
    d = degp_ref[0:1, :] + degp_ref[1:2, :] + 1.0
    out_ref[...] = lax.rsqrt(d)


def _tc_dinv(degp):
    return pl.pallas_call(
        _dinv_body,
        out_shape=jax.ShapeDtypeStruct((1, NPAD), jnp.float32),
    )(degp)


def _pre_body(x_ref, dinv_ref, Wf_ref, bf_ref, W0_ref, out_ref):
    xb = _bn(x_ref[...])
    x1 = jnp.maximum(_mm(xb, Wf_ref[...]) + bf_ref[...], 0.0)
    z = _bn(x1)
    out_ref[...] = _mm(z, W0_ref[...]) * dinv_ref[...]


def _tc_pre(x, dinv, Wf, bf, W0):
    return pl.pallas_call(
        _pre_body,
        out_shape=jax.ShapeDtypeStruct((N, H), jnp.float32),
    )(x, dinv, Wf, bf, W0)


def _mid_body(accp_ref, hs_ref, dinv_ref, b_ref, W_ref, out_ref):
    acc = accp_ref[0, :N, :] + accp_ref[1, :N, :] + hs_ref[...]
    x = jnp.maximum(acc * dinv_ref[...] + b_ref[...], 0.0)
    z = _bn(x)
    out_ref[...] = _mm(z, W_ref[...]) * dinv_ref[...]


def _tc_mid(accp, hs, dinv, b, W):
    return pl.pallas_call(
        _mid_body,
        out_shape=jax.ShapeDtypeStruct((N, H), jnp.float32),
    )(accp, hs, dinv, b, W)


def _att1_body(accp_ref, hs_ref, dinv_ref, b_ref, Wn_ref, bn2_ref,
               natt_ref, xc_att_ref, xo_att_ref):
    acc = accp_ref[0, :N, :] + accp_ref[1, :N, :] + hs_ref[...]
    x = jnp.maximum(acc * dinv_ref[...] + b_ref[...], 0.0)
    logits = _mm(x, Wn_ref[...]) + bn2_ref[...]
    mx = jnp.max(logits, axis=-1, keepdims=True)
    e = jnp.exp(logits - mx)
    a = e / jnp.sum(e, axis=-1, keepdims=True)
    natt_ref[...] = a
    xc_att_ref[...] = a[:, 0:1] * x
    xo_att_ref[...] = a[:, 1:2] * x


def _tc_att1(accp, hs, dinv, b, Wn, bn2):
    return pl.pallas_call(
        _att1_body,
        out_shape=(
            jax.ShapeDtypeStruct((N, 2), jnp.float32),
            jax.ShapeDtypeStruct((N, H), jnp.float32),
            jax.ShapeDtypeStruct((N, H), jnp.float32),
        ),
    )(accp, hs, dinv, b, Wn, bn2)


def _att2_body(xc_ref, xo_ref, dinv_ref, Wc_ref, Wo_ref, hcs_ref, hos_ref):
    hcs_ref[...] = _mm(_bn(xc_ref[...]), Wc_ref[...]) * dinv_ref[...]
    hos_ref[...] = _mm(_bn(xo_ref[...]), Wo_ref[...]) * dinv_ref[...]


def _tc_att2(xc_att, xo_att, dinv, Wc, Wo):
    return pl.pallas_call(
        _att2_body,
        out_shape=(
            jax.ShapeDtypeStruct((N, H), jnp.float32),
            jax.ShapeDtypeStruct((N, H), jnp.float32),
        ),
    )(xc_att, xo_att, dinv, Wc, Wo)


def _pool_body(accc_ref, acco_ref, hcs_ref, hos_ref, dinv_ref,
               bc_ref, bo_ref, batch_ref, sc_ref, so_ref):
    xc = jnp.maximum((accc_ref[0, :N, :] + accc_ref[1, :N, :] + hcs_ref[...])
                     * dinv_ref[...] + bc_ref[...], 0.0)
    xo = jnp.maximum((acco_ref[0, :N, :] + acco_ref[1, :N, :] + hos_ref[...])
                     * dinv_ref[...] + bo_ref[...], 0.0)
    gid = lax.broadcasted_iota(jnp.int32, (G, N), 0)
    onehot = jnp.where(gid == batch_ref[...], 1.0, 0.0)
    sc_ref[...] = _mm(onehot, xc)
    so_ref[...] = _mm(onehot, xo)


def _tc_pool(accc, acco, hcs, hos, dinv, bc, bo, batch_row):
    return pl.pallas_call(
        _pool_body,
        out_shape=(
            jax.ShapeDtypeStruct((G, H), jnp.float32),
            jax.ShapeDtypeStruct((G, H), jnp.float32),
        ),
    )(accc, acco, hcs, hos, dinv, bc, bo, batch_row)


def _readout(h, W1, b1, W2, b2):
    h = _bn(h)
    h = jnp.maximum(_mm(h, W1) + b1, 0.0)
    h = _bn(h)
    l = _mm(h, W2) + b2
    mx = jnp.max(l, axis=-1, keepdims=True)
    return l - mx - jnp.log(jnp.sum(jnp.exp(l - mx), axis=-1, keepdims=True))


def _head_body(sc_ref, so_ref,
               W1c_ref, b1c_ref, W2c_ref, b2c_ref,
               W1o_ref, b1o_ref, W2o_ref, b2o_ref,
               W1co_ref, b1co_ref, W2co_ref, b2co_ref,
               lc_ref, lo_ref, lco_ref):
    sc = sc_ref[...]
    so = so_ref[...]
    lc_ref[...] = _readout(sc, W1c_ref[...], b1c_ref[...],
                           W2c_ref[...], b2c_ref[...])
    lo_ref[...] = _readout(so, W1o_ref[...], b1o_ref[...],
                           W2o_ref[...], b2o_ref[...])
    sco = jnp.concatenate([sc, so], axis=1)
    lco_ref[...] = _readout(sco, W1co_ref[...], b1co_ref[...],
                            W2co_ref[...], b2co_ref[...])


def _tc_head(sc, so, *weights):
    return pl.pallas_call(
        _head_body,
        out_shape=(
            jax.ShapeDtypeStruct((G, C), jnp.float32),
            jax.ShapeDtypeStruct((G, C), jnp.float32),
            jax.ShapeDtypeStruct((G, C), jnp.float32),
        ),
    )(sc, so, *weights)


def kernel(x, edge_index, batch, params):
    p = params
    ei = edge_index

    row = ei[0]
    col = ei[1]
    degp = _sc_deg(col).reshape(NC, NPAD)
    dinv_row = _tc_dinv(degp)
    dinv = dinv_row[0, :N].reshape(N, 1)

    hs0 = _tc_pre(x, dinv, p['W_feat'], p['b_feat'].reshape(1, H), p['Ws'][0])
    acc0 = _sc_prop(hs0, row, col)
    hs1 = _tc_mid(acc0, hs0, dinv, p['bs'][0].reshape(1, H), p['Ws'][1])
    acc1 = _sc_prop(hs1, row, col)
    hs2 = _tc_mid(acc1, hs1, dinv, p['bs'][1].reshape(1, H), p['Ws'][2])
    acc2 = _sc_prop(hs2, row, col)

    Wn = jnp.concatenate([p['Wn_c'], p['Wn_o']], axis=1)
    bn2 = jnp.concatenate([p['bn_c'], p['bn_o']]).reshape(1, 2)
    node_att, xc_att, xo_att = _tc_att1(
        acc2, hs2, dinv, p['bs'][2].reshape(1, H), Wn, bn2)
    hcs, hos = _tc_att2(xc_att, xo_att, dinv, p['W_ctx'], p['W_obj'])

    accc = _sc_prop(hcs, row, col)
    acco = _sc_prop(hos, row, col)

    sc, so = _tc_pool(accc, acco, hcs, hos, dinv,
                      p['b_ctx'].reshape(1, H), p['b_obj'].reshape(1, H),
                      batch.reshape(1, N))

    xc_logis, xo_logis, xco_logis = _tc_head(
        sc, so,
        p['W1c'], p['b1c'].reshape(1, H), p['W2c'], p['b2c'].reshape(1, C),
        p['W1o'], p['b1o'].reshape(1, H), p['W2o'], p['b2o'].reshape(1, C),
        p['W1co'], p['b1co'].reshape(1, H), p['W2co'], p['b2co'].reshape(1, C))

    edge_att_c = jnp.ones((E, 1), jnp.float32)
    return (xc_logis, xo_logis, xco_logis, xc_att, xo_att,
            edge_att_c, node_att)


# speedup vs baseline: 35.1172x; 1.0656x over previous
import functools

import jax
import jax.numpy as jnp
from jax import lax
from jax.experimental import pallas as pl
from jax.experimental.pallas import tpu as pltpu
from jax.experimental.pallas import tpu_sc as plsc

N = 10000
E = 320000
D = 128
H = 128
C = 10
G = 128
EPS = 1e-5

NC = 2
NS = 16
NW = NC * NS
EPT = E // NW
K = 80
NCHUNK = EPT // K
NPAD = 10240
RPT = NPAD // NS
DPT = NPAD // NS

@functools.cache
def _sc_mesh():
    return plsc.VectorSubcoreMesh(core_axis_name="c", subcore_axis_name="s",
                                  num_cores=NC, num_subcores=NS)


def _sc_deg_body(col_hbm, out_hbm, deg_sh, colbuf, ones_v, zbuf, sem0, sem1):
    c = lax.axis_index("c")
    s = lax.axis_index("s")
    wid = c * NS + s
    cpi = pltpu.async_copy(col_hbm.at[wid], colbuf, sem0)
    for i in range(DPT // 16):
        zbuf[pl.ds(i * 16, 16)] = jnp.zeros((16,), jnp.float32)
    for i in range(K // 16):
        ones_v[pl.ds(i * 16, 16)] = jnp.ones((16,), jnp.float32)
    zcp = pltpu.async_copy(zbuf, deg_sh.at[pl.ds(s * DPT, DPT)], sem1)
    cpi.wait()
    zcp.wait()
    plsc.subcore_barrier()

    @pl.loop(0, NCHUNK)
    def _(j):
        pltpu.sync_copy(ones_v, deg_sh.at[colbuf.at[j]], add=True)

    plsc.subcore_barrier()
    pltpu.sync_copy(deg_sh.at[pl.ds(s * DPT, DPT)],
                    out_hbm.at[pl.ds((c * NS + s) * DPT, DPT)])


@functools.cache
def _sc_deg_kernel():
    return pl.kernel(
        _sc_deg_body,
        out_type=jax.ShapeDtypeStruct((NC * NPAD,), jnp.float32),
        mesh=_sc_mesh(),
        scratch_types=[
            pltpu.VMEM_SHARED((NPAD,), jnp.float32),
            pltpu.VMEM((NCHUNK, K), jnp.int32),
            pltpu.VMEM((K,), jnp.float32),
            pltpu.VMEM((DPT,), jnp.float32),
            pltpu.SemaphoreType.DMA,
            pltpu.SemaphoreType.DMA,
        ],
    )


def _sc_deg(col):
    return _sc_deg_kernel()(col.reshape(NW, NCHUNK, K))


def _sc_prop_body(hs_hbm, row_hbm, col_hbm, out_hbm,
                  acc_sh, rowbuf, colbuf, rows0, rows1,
                  sem0, sem1, sem2):
    c = lax.axis_index("c")
    s = lax.axis_index("s")
    wid = c * NS + s
    for r in range(K):
        for q in range(H // 16):
            rows0[r, pl.ds(q * 16, 16)] = jnp.zeros((16,), jnp.float32)

    cpr = pltpu.async_copy(row_hbm.at[wid], rowbuf, sem0)
    cpc = pltpu.async_copy(col_hbm.at[wid], colbuf, sem1)
    zcps = [pltpu.async_copy(rows0, acc_sh.at[pl.ds(s * RPT + i * K, K)], sem2)
            for i in range(RPT // K)]
    cpr.wait()
    cpc.wait()
    for cp in zcps:
        cp.wait()

    plsc.subcore_barrier()

    pltpu.async_copy(hs_hbm.at[rowbuf.at[pl.ds(0, K)]], rows0, sem0).wait()

    @pl.loop(0, NCHUNK - 1, step=2)
    def _(j):
        cp1 = pltpu.async_copy(
            hs_hbm.at[rowbuf.at[pl.ds((j + 1) * K, K)]], rows1, sem1)
        pltpu.sync_copy(rows0, acc_sh.at[colbuf.at[j]], add=True)
        cp0 = pltpu.async_copy(
            hs_hbm.at[rowbuf.at[pl.ds((j + 2) * K, K)]], rows0, sem0)
        cp1.wait()
        pltpu.sync_copy(rows1, acc_sh.at[colbuf.at[j + 1]], add=True)
        cp0.wait()

    pltpu.sync_copy(rows0, acc_sh.at[colbuf.at[NCHUNK - 1]], add=True)

    plsc.subcore_barrier()
    pltpu.sync_copy(acc_sh.at[pl.ds(s * RPT, RPT)],
                    out_hbm.at[c, pl.ds(s * RPT, RPT)])


@functools.cache
def _sc_prop_kernel():
    return pl.kernel(
        _sc_prop_body,
        out_type=jax.ShapeDtypeStruct((NC, NPAD, H), jnp.float32),
        mesh=_sc_mesh(),
        scratch_types=[
            pltpu.VMEM_SHARED((NPAD, H), jnp.float32),
            pltpu.VMEM((EPT,), jnp.int32),
            pltpu.VMEM((NCHUNK, K), jnp.int32),
            pltpu.VMEM((K, H), jnp.float32),
            pltpu.VMEM((K, H), jnp.float32),
            pltpu.SemaphoreType.DMA,
            pltpu.SemaphoreType.DMA,
            pltpu.SemaphoreType.DMA,
        ],
    )


def _sc_prop(hs, row, col):
    return _sc_prop_kernel()(hs.reshape(N, H),
                             row.reshape(NW, EPT),
                             col.reshape(NW, NCHUNK, K))


def _sc_prop2_body(hs2_hbm, row_hbm, col_hbm, out_hbm,
                   acc_sh, rowbuf, colbuf, rows0, rows1,
                   sem0, sem1, sem2):
    c = lax.axis_index("c")
    s = lax.axis_index("s")
    for r in range(K):
        for q in range(H // 16):
            rows0[r, pl.ds(q * 16, 16)] = jnp.zeros((16,), jnp.float32)

    zcps = [pltpu.async_copy(rows0, acc_sh.at[pl.ds(s * RPT + i * K, K)], sem2)
            for i in range(RPT // K)]
    for cp in zcps:
        cp.wait()

    cpr = pltpu.async_copy(row_hbm.at[2 * s], rowbuf, sem0)
    cpc = pltpu.async_copy(col_hbm.at[2 * s], colbuf, sem1)
    cpr.wait()
    cpc.wait()
    plsc.subcore_barrier()
    src = hs2_hbm.at[c]

    for h in range(2):
        pltpu.async_copy(src.at[rowbuf.at[pl.ds(0, K)]], rows0, sem0).wait()

        @pl.loop(0, NCHUNK - 1, step=2)
        def _(j):
            cp1 = pltpu.async_copy(
                src.at[rowbuf.at[pl.ds((j + 1) * K, K)]], rows1, sem1)
            pltpu.sync_copy(rows0, acc_sh.at[colbuf.at[j]], add=True)
            cp0 = pltpu.async_copy(
                src.at[rowbuf.at[pl.ds((j + 2) * K, K)]], rows0, sem0)
            cp1.wait()
            pltpu.sync_copy(rows1, acc_sh.at[colbuf.at[j + 1]], add=True)
            cp0.wait()

        if h == 0:
            cpr = pltpu.async_copy(row_hbm.at[2 * s + 1], rowbuf, sem2)
            pltpu.sync_copy(rows0, acc_sh.at[colbuf.at[NCHUNK - 1]], add=True)
            cpc = pltpu.async_copy(col_hbm.at[2 * s + 1], colbuf, sem1)
            cpr.wait()
            cpc.wait()
        else:
            pltpu.sync_copy(rows0, acc_sh.at[colbuf.at[NCHUNK - 1]], add=True)

    plsc.subcore_barrier()
    pltpu.sync_copy(acc_sh.at[pl.ds(s * RPT, RPT)],
                    out_hbm.at[c, pl.ds(s * RPT, RPT)])


@functools.cache
def _sc_prop2_kernel():
    return pl.kernel(
        _sc_prop2_body,
        out_type=jax.ShapeDtypeStruct((NC, NPAD, H), jnp.float32),
        mesh=_sc_mesh(),
        scratch_types=[
            pltpu.VMEM_SHARED((NPAD, H), jnp.float32),
            pltpu.VMEM((EPT,), jnp.int32),
            pltpu.VMEM((NCHUNK, K), jnp.int32),
            pltpu.VMEM((K, H), jnp.float32),
            pltpu.VMEM((K, H), jnp.float32),
            pltpu.SemaphoreType.DMA,
            pltpu.SemaphoreType.DMA,
            pltpu.SemaphoreType.DMA,
        ],
    )


def _sc_prop2(hs2, row, col):
    return _sc_prop2_kernel()(hs2,
                              row.reshape(NW, EPT),
                              col.reshape(NW, NCHUNK, K))


def _bn(x):
    m = jnp.mean(x, axis=0, keepdims=True)
    v = jnp.mean((x - m) * (x - m), axis=0, keepdims=True)
    return (x - m) * lax.rsqrt(v + EPS) + 1e-4


def _mm(a, b):
    return jnp.dot(a, b, preferred_element_type=jnp.float32)


def _pre_a_body(x_ref, Wf_ref, bf_ref, W0_ref, out_ref):
    xb = _bn(x_ref[...])
    x1 = jnp.maximum(_mm(xb, Wf_ref[...]) + bf_ref[...], 0.0)
    z = _bn(x1)
    out_ref[...] = _mm(z, W0_ref[...])


def _tc_pre_a(x, Wf, bf, W0):
    return pl.pallas_call(
        _pre_a_body,
        out_shape=jax.ShapeDtypeStruct((N, H), jnp.float32),
    )(x, Wf, bf, W0)


def _dinv_body(degp_ref, out_ref):
    d = degp_ref[0:1, :] + degp_ref[1:2, :] + 1.0
    out_ref[...] = lax.rsqrt(d)


def _tc_dinv(degp):
    return pl.pallas_call(
        _dinv_body,
        out_shape=jax.ShapeDtypeStruct((1, NPAD), jnp.float32),
    )(degp)


def _pre_b_body(h0_ref, dinv_ref, hs_ref):
    hs_ref[...] = h0_ref[...] * dinv_ref[...]


def _tc_pre_b(h0, dinv):
    return pl.pallas_call(
        _pre_b_body,
        out_shape=jax.ShapeDtypeStruct((N, H), jnp.float32),
    )(h0, dinv)


def _mid_body(accp_ref, hs_ref, dinv_ref, b_ref, W_ref, out_ref):
    acc = accp_ref[0, :N, :] + accp_ref[1, :N, :] + hs_ref[...]
    x = jnp.maximum(acc * dinv_ref[...] + b_ref[...], 0.0)
    z = _bn(x)
    out_ref[...] = _mm(z, W_ref[...]) * dinv_ref[...]


def _tc_mid(accp, hs, dinv, b, W):
    return pl.pallas_call(
        _mid_body,
        out_shape=jax.ShapeDtypeStruct((N, H), jnp.float32),
    )(accp, hs, dinv, b, W)


def _att_body(accp_ref, hs_ref, dinv_ref, b_ref, Wn_ref, bn2_ref,
              Wc_ref, Wo_ref,
              natt_ref, xc_att_ref, xo_att_ref, hs2_ref):
    acc = accp_ref[0, :N, :] + accp_ref[1, :N, :] + hs_ref[...]
    x = jnp.maximum(acc * dinv_ref[...] + b_ref[...], 0.0)
    logits = _mm(x, Wn_ref[...]) + bn2_ref[...]
    mx = jnp.max(logits, axis=-1, keepdims=True)
    e = jnp.exp(logits - mx)
    a = e / jnp.sum(e, axis=-1, keepdims=True)
    natt_ref[...] = a
    xc = a[:, 0:1] * x
    xo = a[:, 1:2] * x
    xc_att_ref[...] = xc
    xo_att_ref[...] = xo
    hs2_ref[0, :, :] = _mm(_bn(xc), Wc_ref[...]) * dinv_ref[...]
    hs2_ref[1, :, :] = _mm(_bn(xo), Wo_ref[...]) * dinv_ref[...]


def _tc_att(accp, hs, dinv, b, Wn, bn2, Wc, Wo):
    return pl.pallas_call(
        _att_body,
        out_shape=(
            jax.ShapeDtypeStruct((N, 2), jnp.float32),
            jax.ShapeDtypeStruct((N, H), jnp.float32),
            jax.ShapeDtypeStruct((N, H), jnp.float32),
            jax.ShapeDtypeStruct((2, N, H), jnp.float32),
        ),
        compiler_params=pltpu.CompilerParams(
            vmem_limit_bytes=100 * 1024 * 1024),
    )(accp, hs, dinv, b, Wn, bn2, Wc, Wo)


def _pool_body(acc2_ref, hs2_ref, dinv_ref,
               bc_ref, bo_ref, batch_ref,
               W1c_ref, b1c_ref, W2c_ref, b2c_ref,
               W1o_ref, b1o_ref, W2o_ref, b2o_ref,
               W1co_ref, b1co_ref, W2co_ref, b2co_ref,
               lc_ref, lo_ref, lco_ref):
    xc = jnp.maximum((acc2_ref[0, :N, :] + hs2_ref[0, :, :])
                     * dinv_ref[...] + bc_ref[...], 0.0)
    xo = jnp.maximum((acc2_ref[1, :N, :] + hs2_ref[1, :, :])
                     * dinv_ref[...] + bo_ref[...], 0.0)
    gid = lax.broadcasted_iota(jnp.int32, (G, N), 0)
    onehot = jnp.where(gid == batch_ref[...], 1.0, 0.0)
    sc = _mm(onehot, xc)
    so = _mm(onehot, xo)
    lc_ref[...] = _readout(sc, W1c_ref[...], b1c_ref[...],
                           W2c_ref[...], b2c_ref[...])
    lo_ref[...] = _readout(so, W1o_ref[...], b1o_ref[...],
                           W2o_ref[...], b2o_ref[...])
    sco = jnp.concatenate([sc, so], axis=1)
    lco_ref[...] = _readout(sco, W1co_ref[...], b1co_ref[...],
                            W2co_ref[...], b2co_ref[...])


def _tc_pool(acc2, hs2, dinv, bc, bo, batch_row, *weights):
    return pl.pallas_call(
        _pool_body,
        out_shape=(
            jax.ShapeDtypeStruct((G, C), jnp.float32),
            jax.ShapeDtypeStruct((G, C), jnp.float32),
            jax.ShapeDtypeStruct((G, C), jnp.float32),
        ),
    )(acc2, hs2, dinv, bc, bo, batch_row, *weights)


def _readout(h, W1, b1, W2, b2):
    h = _bn(h)
    h = jnp.maximum(_mm(h, W1) + b1, 0.0)
    h = _bn(h)
    l = _mm(h, W2) + b2
    mx = jnp.max(l, axis=-1, keepdims=True)
    return l - mx - jnp.log(jnp.sum(jnp.exp(l - mx), axis=-1, keepdims=True))


def kernel(x, edge_index, batch, params):
    p = params
    ei = edge_index

    row = ei[0]
    col = ei[1]
    degp = _sc_deg(col).reshape(NC, NPAD)
    h0 = _tc_pre_a(x, p['W_feat'], p['b_feat'].reshape(1, H), p['Ws'][0])
    dinv = _tc_dinv(degp)[0, :N].reshape(N, 1)
    hs0 = _tc_pre_b(h0, dinv)
    acc0 = _sc_prop(hs0, row, col)
    hs1 = _tc_mid(acc0, hs0, dinv, p['bs'][0].reshape(1, H), p['Ws'][1])
    acc1 = _sc_prop(hs1, row, col)
    hs2 = _tc_mid(acc1, hs1, dinv, p['bs'][1].reshape(1, H), p['Ws'][2])
    acc2 = _sc_prop(hs2, row, col)

    Wn = jnp.concatenate([p['Wn_c'], p['Wn_o']], axis=1)
    bn2 = jnp.concatenate([p['bn_c'], p['bn_o']]).reshape(1, 2)
    node_att, xc_att, xo_att, hs2co = _tc_att(
        acc2, hs2, dinv, p['bs'][2].reshape(1, H), Wn, bn2,
        p['W_ctx'], p['W_obj'])

    acc2co = _sc_prop2(hs2co, row, col)

    xc_logis, xo_logis, xco_logis = _tc_pool(
        acc2co, hs2co, dinv,
        p['b_ctx'].reshape(1, H), p['b_obj'].reshape(1, H),
        batch.reshape(1, N),
        p['W1c'], p['b1c'].reshape(1, H), p['W2c'], p['b2c'].reshape(1, C),
        p['W1o'], p['b1o'].reshape(1, H), p['W2o'], p['b2o'].reshape(1, C),
        p['W1co'], p['b1co'].reshape(1, H), p['W2co'], p['b2co'].reshape(1, C))

    edge_att_c = jnp.ones((E, 1), jnp.float32)
    return (xc_logis, xo_logis, xco_logis, xc_att, xo_att,
            edge_att_c, node_att)
